# Initial kernel scaffold; baseline (speedup 1.0000x reference)
import jax
import jax.numpy as jnp
from jax.experimental import pallas as pl


def kernel(featmap_a, featmap_q, mask_a_logits, mask_q_logits, corrs, valid, anchor_rgb, anchor_mask, query_mask):
    raise NotImplementedError("write your pallas kernel here")


# speedup vs baseline: 1.8562x; 1.8562x over previous
import functools

import jax
import jax.numpy as jnp
from jax.experimental import pallas as pl
from jax.experimental.pallas import tpu as pltpu

POS_MARGIN = 0.1
NEG_MARGIN = 1.4
NEG_KERNEL = 9.0
MASK_TH = 0.5


def _main_kernel(fm_a_ref, fm_q_ref, corr_ref, valid_ref,
                 idxa_ref, idxq_ref, pos_ref, nega_ref, negq_ref,
                 fmn_a, fmn_q, fmn16_a, fmn16_q, *, R, D, FH, FW):
    HW = FH * FW
    b = pl.program_id(0)
    t = pl.program_id(1)

    @pl.when(t == 0)
    def _():
        fa = fm_a_ref[0]
        na = jnp.sqrt(jnp.sum(fa * fa, axis=0, keepdims=True))
        fmn_a[...] = fa / jnp.maximum(na, 1e-8)
        fq = fm_q_ref[0]
        nq = jnp.sqrt(jnp.sum(fq * fq, axis=0, keepdims=True))
        fmn_q[...] = fq / jnp.maximum(nq, 1e-8)
        fmn16_a[...] = fmn_a[...].astype(jnp.bfloat16)
        fmn16_q[...] = fmn_q[...].astype(jnp.bfloat16)

    @pl.when(jnp.logical_and(b == 0, t == 0))
    def _():
        pos_ref[...] = jnp.zeros_like(pos_ref)
        nega_ref[...] = jnp.zeros_like(nega_ref)
        negq_ref[...] = jnp.zeros_like(negq_ref)

    vm = (valid_ref[0, 0, 0] == 1).astype(jnp.float32)
    corr = corr_ref[0, 0]
    gt = jnp.clip(corr // 8, 0, FH - 1)

    col = jax.lax.broadcasted_iota(jnp.int32, (R, HW), 1)
    yyf = (col // FW).astype(jnp.float32)
    xxf = (col % FW).astype(jnp.float32)

    def one_side(fmn, fmn16, y, x):
        lin = y * FW + x
        oh = (col == lin).astype(jnp.float32)
        posn = jax.lax.dot_general(
            oh, fmn[...], (((1,), (1,)), ((), ())),
            precision=jax.lax.Precision.HIGHEST,
            preferred_element_type=jnp.float32) * vm
        m = jax.lax.dot_general(
            posn.astype(jnp.bfloat16), fmn16[...], (((1,), (0,)), ((), ())),
            precision=jax.lax.Precision.DEFAULT,
            preferred_element_type=jnp.float32)
        fd = 0.5 * (1.0 - m)
        yf = y.astype(jnp.float32)
        xf = x.astype(jnp.float32)
        d2 = (yf * yf + xf * xf) + (yyf * yyf + xxf * xxf) \
            - 2.0 * (yf * yyf + xf * xxf)
        pix = jnp.sqrt(jnp.maximum(d2, 0.0))
        fdp = fd + 1e6 * jax.nn.relu(NEG_KERNEL - pix)
        minv = jnp.min(fdp, axis=1, keepdims=True)
        idx = jnp.min(jnp.where(fdp == minv, col, HW), axis=1,
                      keepdims=True)
        dist_neg = jnp.sum(jnp.where(col == idx, fd, 0.0), axis=1,
                           keepdims=True)
        ny = (idx // FW).astype(jnp.float32) * vm
        nx = (idx % FW).astype(jnp.float32) * vm
        return posn, dist_neg, jnp.concatenate([ny, nx], axis=1)

    posn_a, dneg_a, oidx_a = one_side(fmn_a, fmn16_a, gt[:, 0:1], gt[:, 1:2])
    posn_q, dneg_q, oidx_q = one_side(fmn_q, fmn16_q, gt[:, 2:3], gt[:, 3:4])
    idxa_ref[0, 0] = oidx_a
    idxq_ref[0, 0] = oidx_q

    dist_pos = 0.5 * (1.0 - jnp.sum(posn_a * posn_q, axis=1, keepdims=True))
    pos_ref[...] = pos_ref[...] + vm * jnp.sum(jax.nn.relu(dist_pos - POS_MARGIN))
    nega_ref[...] = nega_ref[...] + vm * jnp.sum(jax.nn.relu(NEG_MARGIN - dneg_a))
    negq_ref[...] = negq_ref[...] + vm * jnp.sum(jax.nn.relu(NEG_MARGIN - dneg_q))


def _mask_kernel(la_ref, lq_ref, ma_ref, mq_ref,
                 pma_ref, pmq_ref, bcea_ref, bceq_ref, ioua_ref, iouq_ref,
                 *, B, FH, FW):
    b = pl.program_id(0)

    @pl.when(b == 0)
    def _():
        bcea_ref[...] = jnp.zeros_like(bcea_ref)
        bceq_ref[...] = jnp.zeros_like(bceq_ref)
        ioua_ref[...] = jnp.zeros_like(ioua_ref)
        iouq_ref[...] = jnp.zeros_like(iouq_ref)

    last = b == B - 1

    def one(l_ref, m_ref, pm_ref, bce_ref, iou_ref):
        x = l_ref[0]
        z = m_ref[0].astype(jnp.float32)
        bce = jnp.sum(jax.nn.relu(x) - x * z + jnp.log1p(jnp.exp(-jnp.abs(x))))
        tot = bce_ref[...] + bce
        bce_ref[...] = jnp.where(last, tot / (B * FH * FW), tot)
        pred = (jax.nn.sigmoid(x) > MASK_TH).astype(jnp.int32)
        pm_ref[0] = pred
        pf = pred.astype(jnp.float32)
        inter = jnp.sum(z * pf)
        union = jnp.sum(jnp.clip(z + pf, 0.0, 1.0))
        itot = iou_ref[...] + inter / (union + 1e-6)
        iou_ref[...] = jnp.where(last, itot / B, itot)

    one(la_ref, ma_ref, pma_ref, bcea_ref, ioua_ref)
    one(lq_ref, mq_ref, pmq_ref, bceq_ref, iouq_ref)


def kernel(featmap_a, featmap_q, mask_a_logits, mask_q_logits, corrs, valid,
           anchor_rgb, anchor_mask, query_mask):
    B, D, FH, FW = featmap_a.shape
    HW = FH * FW
    N = corrs.shape[1]
    R = 256
    NT = N // R

    fa = featmap_a.reshape(B, D, HW)
    fq = featmap_q.reshape(B, D, HW)
    corr4 = corrs.reshape(B, NT, R, 4)
    valid3 = valid.reshape(B, 1, 1)

    main = pl.pallas_call(
        functools.partial(_main_kernel, R=R, D=D, FH=FH, FW=FW),
        grid=(B, NT),
        in_specs=[
            pl.BlockSpec((1, D, HW), lambda b, t: (b, 0, 0)),
            pl.BlockSpec((1, D, HW), lambda b, t: (b, 0, 0)),
            pl.BlockSpec((1, 1, R, 4), lambda b, t: (b, t, 0, 0)),
            pl.BlockSpec((1, 1, 1), lambda b, t: (b, 0, 0)),
        ],
        out_specs=[
            pl.BlockSpec((1, 1, R, 2), lambda b, t: (b, t, 0, 0)),
            pl.BlockSpec((1, 1, R, 2), lambda b, t: (b, t, 0, 0)),
            pl.BlockSpec((1, 1), lambda b, t: (0, 0)),
            pl.BlockSpec((1, 1), lambda b, t: (0, 0)),
            pl.BlockSpec((1, 1), lambda b, t: (0, 0)),
        ],
        out_shape=[
            jax.ShapeDtypeStruct((B, NT, R, 2), jnp.float32),
            jax.ShapeDtypeStruct((B, NT, R, 2), jnp.float32),
            jax.ShapeDtypeStruct((1, 1), jnp.float32),
            jax.ShapeDtypeStruct((1, 1), jnp.float32),
            jax.ShapeDtypeStruct((1, 1), jnp.float32),
        ],
        scratch_shapes=[
            pltpu.VMEM((D, HW), jnp.float32),
            pltpu.VMEM((D, HW), jnp.float32),
            pltpu.VMEM((D, HW), jnp.bfloat16),
            pltpu.VMEM((D, HW), jnp.bfloat16),
        ],
    )
    idxa, idxq, pos_s, nega_s, negq_s = main(fa, fq, corr4, valid3)

    ma = anchor_mask[:, ::8, ::8]
    mq = query_mask[:, ::8, ::8]
    la = mask_a_logits.reshape(B, FH, FW)
    lq = mask_q_logits.reshape(B, FH, FW)
    mask_out = pl.pallas_call(
        functools.partial(_mask_kernel, B=B, FH=FH, FW=FW),
        grid=(B,),
        in_specs=[
            pl.BlockSpec((1, FH, FW), lambda b: (b, 0, 0)),
            pl.BlockSpec((1, FH, FW), lambda b: (b, 0, 0)),
            pl.BlockSpec((1, FH, FW), lambda b: (b, 0, 0)),
            pl.BlockSpec((1, FH, FW), lambda b: (b, 0, 0)),
        ],
        out_specs=[
            pl.BlockSpec((1, FH, FW), lambda b: (b, 0, 0)),
            pl.BlockSpec((1, FH, FW), lambda b: (b, 0, 0)),
            pl.BlockSpec((1, 1), lambda b: (0, 0)),
            pl.BlockSpec((1, 1), lambda b: (0, 0)),
            pl.BlockSpec((1, 1), lambda b: (0, 0)),
            pl.BlockSpec((1, 1), lambda b: (0, 0)),
        ],
        out_shape=[
            jax.ShapeDtypeStruct((B, FH, FW), jnp.int32),
            jax.ShapeDtypeStruct((B, FH, FW), jnp.int32),
            jax.ShapeDtypeStruct((1, 1), jnp.float32),
            jax.ShapeDtypeStruct((1, 1), jnp.float32),
            jax.ShapeDtypeStruct((1, 1), jnp.float32),
            jax.ShapeDtypeStruct((1, 1), jnp.float32),
        ],
    )
    pm_a, pm_q, bce_a, bce_q, iou_a, iou_q = mask_out(la, lq, ma, mq)

    vmask = (valid == 1).astype(jnp.float32)
    cnt = jnp.sum(vmask)
    denom = jnp.maximum(cnt, 1.0)
    pos_loss = jnp.where(cnt > 0, (pos_s[0, 0] / N) / denom, 0.0)
    neg_loss_a = jnp.where(cnt > 0, (nega_s[0, 0] / N) / denom, 0.0)
    neg_loss_q = jnp.where(cnt > 0, (negq_s[0, 0] / N) / denom, 0.0)
    losses = jnp.stack([0.5 * (bce_a[0, 0] + bce_q[0, 0]), pos_loss,
                        0.5 * (neg_loss_a + neg_loss_q)])
    return (losses,
            idxa.reshape(B, N, 2), idxq.reshape(B, N, 2),
            pm_a, pm_q,
            iou_a[0, 0], iou_q[0, 0])


# speedup vs baseline: 4.8208x; 2.5972x over previous
import functools

import jax
import jax.numpy as jnp
from jax.experimental import pallas as pl
from jax.experimental.pallas import tpu as pltpu

POS_MARGIN = 0.1
NEG_MARGIN = 1.4
NEG_KERNEL = 9.0
MASK_TH = 0.5


def _main_kernel(fm_a_ref, fm_q_ref, corr_ref, valid_ref,
                 idxa_ref, idxq_ref, pos_ref, nega_ref, negq_ref,
                 fmn_a, fmn_q, fmn16_a, fmn16_q, *, R, D, FH, FW):
    HW = FH * FW
    b = pl.program_id(0)
    t = pl.program_id(1)

    @pl.when(t == 0)
    def _():
        fa = fm_a_ref[0]
        na = jnp.sqrt(jnp.sum(fa * fa, axis=0, keepdims=True))
        fmn_a[...] = fa / jnp.maximum(na, 1e-8)
        fq = fm_q_ref[0]
        nq = jnp.sqrt(jnp.sum(fq * fq, axis=0, keepdims=True))
        fmn_q[...] = fq / jnp.maximum(nq, 1e-8)
        fmn16_a[...] = fmn_a[...].astype(jnp.bfloat16)
        fmn16_q[...] = fmn_q[...].astype(jnp.bfloat16)

    @pl.when(jnp.logical_and(b == 0, t == 0))
    def _():
        pos_ref[...] = jnp.zeros_like(pos_ref)
        nega_ref[...] = jnp.zeros_like(nega_ref)
        negq_ref[...] = jnp.zeros_like(negq_ref)

    vm = (valid_ref[0, 0, 0] == 1).astype(jnp.float32)
    corr = corr_ref[0, 0]
    gt = jnp.clip(corr // 8, 0, FH - 1)

    col = jax.lax.broadcasted_iota(jnp.int32, (R, HW), 1)
    yy16 = (col // FW).astype(jnp.bfloat16)
    xx16 = (col % FW).astype(jnp.bfloat16)

    def one_side(fmn16, y, x):
        lin = y * FW + x
        oh = (col == lin).astype(jnp.bfloat16)
        posn = jax.lax.dot_general(
            oh, fmn16[...], (((1,), (1,)), ((), ())),
            precision=jax.lax.Precision.DEFAULT,
            preferred_element_type=jnp.float32) * vm
        mh = jax.lax.dot_general(
            (posn * -0.5).astype(jnp.bfloat16), fmn16[...],
            (((1,), (0,)), ((), ())),
            precision=jax.lax.Precision.DEFAULT,
            preferred_element_type=jnp.float32)
        fd = 0.5 + mh
        y16 = y.astype(jnp.bfloat16)
        x16 = x.astype(jnp.bfloat16)
        dy = y16 - yy16
        dx = x16 - xx16
        d2 = dy * dy + dx * dx
        fdp = jnp.where(d2 < jnp.bfloat16(81.0), 1e9, fd)
        minv = jnp.min(fdp, axis=1, keepdims=True)
        idx = jnp.argmin(fdp, axis=1).astype(jnp.int32)[:, None]
        dist_neg = minv
        idx_f = idx.astype(jnp.float32)
        ny = jnp.floor(idx_f * jnp.float32(1.0 / FW))
        nx = idx_f - ny * jnp.float32(FW)
        return posn, dist_neg, jnp.concatenate([ny * vm, nx * vm], axis=1)

    posn_a, dneg_a, oidx_a = one_side(fmn16_a, gt[:, 0:1], gt[:, 1:2])
    posn_q, dneg_q, oidx_q = one_side(fmn16_q, gt[:, 2:3], gt[:, 3:4])
    idxa_ref[0, 0] = oidx_a
    idxq_ref[0, 0] = oidx_q

    dist_pos = 0.5 * (1.0 - jnp.sum(posn_a * posn_q, axis=1, keepdims=True))
    pos_ref[...] = pos_ref[...] + vm * jnp.sum(jax.nn.relu(dist_pos - POS_MARGIN))
    nega_ref[...] = nega_ref[...] + vm * jnp.sum(jax.nn.relu(NEG_MARGIN - dneg_a))
    negq_ref[...] = negq_ref[...] + vm * jnp.sum(jax.nn.relu(NEG_MARGIN - dneg_q))


def _mask_kernel(la_ref, lq_ref, ma_ref, mq_ref,
                 pma_ref, pmq_ref, bcea_ref, bceq_ref, ioua_ref, iouq_ref,
                 *, B, FH, FW):
    b = pl.program_id(0)

    @pl.when(b == 0)
    def _():
        bcea_ref[...] = jnp.zeros_like(bcea_ref)
        bceq_ref[...] = jnp.zeros_like(bceq_ref)
        ioua_ref[...] = jnp.zeros_like(ioua_ref)
        iouq_ref[...] = jnp.zeros_like(iouq_ref)

    last = b == B - 1

    def one(l_ref, m_ref, pm_ref, bce_ref, iou_ref):
        x = l_ref[0]
        z = m_ref[0].astype(jnp.float32)
        bce = jnp.sum(jax.nn.relu(x) - x * z + jnp.log1p(jnp.exp(-jnp.abs(x))))
        tot = bce_ref[...] + bce
        bce_ref[...] = jnp.where(last, tot / (B * FH * FW), tot)
        pred = (jax.nn.sigmoid(x) > MASK_TH).astype(jnp.int32)
        pm_ref[0] = pred
        pf = pred.astype(jnp.float32)
        inter = jnp.sum(z * pf)
        union = jnp.sum(jnp.clip(z + pf, 0.0, 1.0))
        itot = iou_ref[...] + inter / (union + 1e-6)
        iou_ref[...] = jnp.where(last, itot / B, itot)

    one(la_ref, ma_ref, pma_ref, bcea_ref, ioua_ref)
    one(lq_ref, mq_ref, pmq_ref, bceq_ref, iouq_ref)


def kernel(featmap_a, featmap_q, mask_a_logits, mask_q_logits, corrs, valid,
           anchor_rgb, anchor_mask, query_mask):
    B, D, FH, FW = featmap_a.shape
    HW = FH * FW
    N = corrs.shape[1]
    R = 1024
    NT = N // R

    fa = featmap_a.reshape(B, D, HW)
    fq = featmap_q.reshape(B, D, HW)
    corr4 = corrs.reshape(B, NT, R, 4)
    valid3 = valid.reshape(B, 1, 1)

    main = pl.pallas_call(
        functools.partial(_main_kernel, R=R, D=D, FH=FH, FW=FW),
        grid=(B, NT),
        in_specs=[
            pl.BlockSpec((1, D, HW), lambda b, t: (b, 0, 0)),
            pl.BlockSpec((1, D, HW), lambda b, t: (b, 0, 0)),
            pl.BlockSpec((1, 1, R, 4), lambda b, t: (b, t, 0, 0)),
            pl.BlockSpec((1, 1, 1), lambda b, t: (b, 0, 0)),
        ],
        out_specs=[
            pl.BlockSpec((1, 1, R, 2), lambda b, t: (b, t, 0, 0)),
            pl.BlockSpec((1, 1, R, 2), lambda b, t: (b, t, 0, 0)),
            pl.BlockSpec((1, 1), lambda b, t: (0, 0)),
            pl.BlockSpec((1, 1), lambda b, t: (0, 0)),
            pl.BlockSpec((1, 1), lambda b, t: (0, 0)),
        ],
        out_shape=[
            jax.ShapeDtypeStruct((B, NT, R, 2), jnp.float32),
            jax.ShapeDtypeStruct((B, NT, R, 2), jnp.float32),
            jax.ShapeDtypeStruct((1, 1), jnp.float32),
            jax.ShapeDtypeStruct((1, 1), jnp.float32),
            jax.ShapeDtypeStruct((1, 1), jnp.float32),
        ],
        scratch_shapes=[
            pltpu.VMEM((D, HW), jnp.float32),
            pltpu.VMEM((D, HW), jnp.float32),
            pltpu.VMEM((D, HW), jnp.bfloat16),
            pltpu.VMEM((D, HW), jnp.bfloat16),
        ],
    )
    idxa, idxq, pos_s, nega_s, negq_s = main(fa, fq, corr4, valid3)

    ma = anchor_mask[:, ::8, ::8]
    mq = query_mask[:, ::8, ::8]
    la = mask_a_logits.reshape(B, FH, FW)
    lq = mask_q_logits.reshape(B, FH, FW)
    mask_out = pl.pallas_call(
        functools.partial(_mask_kernel, B=B, FH=FH, FW=FW),
        grid=(B,),
        in_specs=[
            pl.BlockSpec((1, FH, FW), lambda b: (b, 0, 0)),
            pl.BlockSpec((1, FH, FW), lambda b: (b, 0, 0)),
            pl.BlockSpec((1, FH, FW), lambda b: (b, 0, 0)),
            pl.BlockSpec((1, FH, FW), lambda b: (b, 0, 0)),
        ],
        out_specs=[
            pl.BlockSpec((1, FH, FW), lambda b: (b, 0, 0)),
            pl.BlockSpec((1, FH, FW), lambda b: (b, 0, 0)),
            pl.BlockSpec((1, 1), lambda b: (0, 0)),
            pl.BlockSpec((1, 1), lambda b: (0, 0)),
            pl.BlockSpec((1, 1), lambda b: (0, 0)),
            pl.BlockSpec((1, 1), lambda b: (0, 0)),
        ],
        out_shape=[
            jax.ShapeDtypeStruct((B, FH, FW), jnp.int32),
            jax.ShapeDtypeStruct((B, FH, FW), jnp.int32),
            jax.ShapeDtypeStruct((1, 1), jnp.float32),
            jax.ShapeDtypeStruct((1, 1), jnp.float32),
            jax.ShapeDtypeStruct((1, 1), jnp.float32),
            jax.ShapeDtypeStruct((1, 1), jnp.float32),
        ],
    )
    pm_a, pm_q, bce_a, bce_q, iou_a, iou_q = mask_out(la, lq, ma, mq)

    vmask = (valid == 1).astype(jnp.float32)
    cnt = jnp.sum(vmask)
    denom = jnp.maximum(cnt, 1.0)
    pos_loss = jnp.where(cnt > 0, (pos_s[0, 0] / N) / denom, 0.0)
    neg_loss_a = jnp.where(cnt > 0, (nega_s[0, 0] / N) / denom, 0.0)
    neg_loss_q = jnp.where(cnt > 0, (negq_s[0, 0] / N) / denom, 0.0)
    losses = jnp.stack([0.5 * (bce_a[0, 0] + bce_q[0, 0]), pos_loss,
                        0.5 * (neg_loss_a + neg_loss_q)])
    return (losses,
            idxa.reshape(B, N, 2), idxq.reshape(B, N, 2),
            pm_a, pm_q,
            iou_a[0, 0], iou_q[0, 0])
